# Initial kernel scaffold; baseline (speedup 1.0000x reference)
import functools

import jax
import jax.numpy as jnp
from jax import lax
from jax.experimental import pallas as pl
from jax.experimental.pallas import tpu as pltpu, tpu_sc as plsc

NUM_GRAPHS = 256
NPG = 113
N = NUM_GRAPHS * NPG
E = N * 16
NC, NS = 2, 16
NW = NC * NS
CH = 128
ER = E // CH
RPT32 = ER // NW
RPT16 = ER // NS
NPS = N // NS

_mesh = plsc.VectorSubcoreMesh(core_axis_name="c", subcore_axis_name="s")


def _lrelu(x):
    return jnp.where(x > 0, x, 0.2 * x)


@functools.partial(
    pl.kernel,
    out_type=jax.ShapeDtypeStruct((NC, N), jnp.float32),
    mesh=_mesh,
    scratch_types=[
        pltpu.VMEM((RPT32, CH), jnp.int32),
        pltpu.VMEM((RPT32, CH), jnp.float32),
        pltpu.VMEM_SHARED((N,), jnp.float32),
    ],
)
def _k1_deg(dst_hbm, w_hbm, z1_hbm, out_hbm, dstb, wb, acc):
    c = lax.axis_index("c")
    s = lax.axis_index("s")
    wid = c * NS + s
    pltpu.sync_copy(z1_hbm, acc.at[pl.ds(s * NPS, NPS)])
    plsc.subcore_barrier()
    pltpu.sync_copy(dst_hbm.at[pl.ds(wid * RPT32, RPT32)], dstb)
    pltpu.sync_copy(w_hbm.at[pl.ds(wid * RPT32, RPT32)], wb)

    def chunk(j, _):
        pltpu.sync_copy(wb.at[j], acc.at[dstb.at[j]], add=True)
        return ()

    lax.fori_loop(0, RPT32, chunk, ())
    plsc.subcore_barrier()
    pltpu.sync_copy(acc.at[pl.ds(s * NPS, NPS)],
                    out_hbm.at[c].at[pl.ds(s * NPS, NPS)])


@functools.partial(
    pl.kernel,
    out_type=jax.ShapeDtypeStruct((NC, N, 16), jnp.float32),
    mesh=_mesh,
    scratch_types=[
        pltpu.VMEM((RPT32, CH), jnp.int32),
        pltpu.VMEM((RPT32, CH), jnp.int32),
        pltpu.VMEM((RPT32, CH), jnp.float32),
        pltpu.VMEM((CH, 16), jnp.float32),
        pltpu.VMEM_SHARED((N, 16), jnp.float32),
        pltpu.SemaphoreType.DMA,
    ],
)
def _k2_conv1(src_hbm, dst_hbm, w_hbm, tbl_hbm, z16_hbm, out_hbm,
              srcb, dstb, wb, rows, acc, sem):
    c = lax.axis_index("c")
    s = lax.axis_index("s")
    wid = c * NS + s
    pltpu.sync_copy(z16_hbm, acc.at[pl.ds(s * NPS, NPS)])
    plsc.subcore_barrier()
    pltpu.sync_copy(src_hbm.at[pl.ds(wid * RPT32, RPT32)], srcb)
    pltpu.sync_copy(dst_hbm.at[pl.ds(wid * RPT32, RPT32)], dstb)
    pltpu.sync_copy(w_hbm.at[pl.ds(wid * RPT32, RPT32)], wb)

    def chunk(j, _):
        pltpu.async_copy(tbl_hbm.at[srcb.at[j]], rows, sem).wait()

        def scale(e, _):
            rows[e, :] = rows[e, :] * wb[j, e]
            return ()

        lax.fori_loop(0, CH, scale, (), unroll=4)
        pltpu.sync_copy(rows, acc.at[dstb.at[j]], add=True)
        return ()

    lax.fori_loop(0, RPT32, chunk, ())
    plsc.subcore_barrier()
    pltpu.sync_copy(acc.at[pl.ds(s * NPS, NPS)],
                    out_hbm.at[c].at[pl.ds(s * NPS, NPS)])


@functools.partial(
    pl.kernel,
    out_type=jax.ShapeDtypeStruct((NC, N, 64), jnp.float32),
    mesh=_mesh,
    scratch_types=[
        pltpu.VMEM((RPT16, CH), jnp.int32),
        pltpu.VMEM((RPT16, CH), jnp.int32),
        pltpu.VMEM((RPT16, CH), jnp.float32),
        pltpu.VMEM((CH, 64), jnp.float32),
        pltpu.VMEM_SHARED((N, 64), jnp.float32),
        pltpu.SemaphoreType.DMA,
    ],
)
def _k3_conv2(src_hbm, dst_hbm, w_hbm, tbl_hbm, g0_hbm, g1_hbm, out_hbm,
              srcb, dstb, wb, rows, acc, sem):
    c = lax.axis_index("c")
    s = lax.axis_index("s")

    @pl.when(c == 0)
    def _():
        pltpu.sync_copy(g0_hbm.at[pl.ds(s * NPS, NPS)],
                        acc.at[pl.ds(s * NPS, NPS)])

    @pl.when(c == 1)
    def _():
        pltpu.sync_copy(g1_hbm.at[pl.ds(s * NPS, NPS)],
                        acc.at[pl.ds(s * NPS, NPS)])

    plsc.subcore_barrier()
    pltpu.sync_copy(src_hbm.at[pl.ds(s * RPT16, RPT16)], srcb)
    pltpu.sync_copy(dst_hbm.at[pl.ds(s * RPT16, RPT16)], dstb)
    pltpu.sync_copy(w_hbm.at[pl.ds(s * RPT16, RPT16)], wb)

    def to_gidx(k, _):
        for b in range(8):
            v = srcb[k, pl.ds(b * 16, 16)]
            srcb[k, pl.ds(b * 16, 16)] = v * 2 + c
        return ()

    lax.fori_loop(0, RPT16, to_gidx, ())

    def chunk(j, _):
        pltpu.async_copy(tbl_hbm.at[srcb.at[j]], rows, sem).wait()

        def scale(e, _):
            w = wb[j, e]
            for q in range(4):
                rows[e, pl.ds(q * 16, 16)] = rows[e, pl.ds(q * 16, 16)] * w
            return ()

        lax.fori_loop(0, CH, scale, (), unroll=2)
        pltpu.sync_copy(rows, acc.at[dstb.at[j]], add=True)
        return ()

    lax.fori_loop(0, RPT16, chunk, ())
    plsc.subcore_barrier()
    pltpu.sync_copy(acc.at[pl.ds(s * NPS, NPS)],
                    out_hbm.at[c].at[pl.ds(s * NPS, NPS)])


def _t1_body(dp_ref, xp_ref, sc_ref, dis_ref, tbl_ref):
    d = dp_ref[0, 0, :] + dp_ref[0, 1, :] + 1.0
    dis = lax.rsqrt(d)
    dis_ref[0, 0, :] = dis
    tbl_ref[0] = xp_ref[0] * sc_ref[0] * dis[:, None]


def _t2_body(a0_ref, a1_ref, t1_ref, dis_ref, w1_ref, b1_ref,
             gt_ref, g0_ref, g1_ref):
    m1 = (a0_ref[0] + a1_ref[0] + t1_ref[0]) * dis_ref[0]
    h = lax.dot_general(m1, w1_ref[...], (((1,), (1,)), ((), ())),
                        preferred_element_type=jnp.float32) + b1_ref[...]
    g = _lrelu(h) * dis_ref[0]
    gt_ref[0] = g
    g0_ref[0] = g[:, :64]
    g1_ref[0] = g[:, 64:]


def _t3_body(m0_ref, m1_ref, dis_ref, w2a_ref, w2b_ref, b2_ref, z_ref):
    ma = m0_ref[0] * dis_ref[0]
    mb = m1_ref[0] * dis_ref[0]
    h = (lax.dot_general(ma, w2a_ref[...], (((1,), (1,)), ((), ())),
                         preferred_element_type=jnp.float32)
         + lax.dot_general(mb, w2b_ref[...], (((1,), (1,)), ((), ())),
                           preferred_element_type=jnp.float32)
         + b2_ref[...])
    z_ref[0, 0, :] = jnp.max(_lrelu(h), axis=0)


def _t4_body(z_ref, p1w_ref, p1b_ref, g_ref, b_ref, p2w_ref, p2b_ref, o_ref):
    p = lax.dot_general(z_ref[...], p1w_ref[...], (((1,), (1,)), ((), ())),
                        preferred_element_type=jnp.float32) + p1b_ref[...]
    mean = jnp.mean(p, axis=0, keepdims=True)
    var = jnp.mean((p - mean) ** 2, axis=0, keepdims=True)
    p = (p - mean) / jnp.sqrt(var + 1e-5) * g_ref[...] + b_ref[...]
    p = jnp.maximum(p, 0.0)
    o = lax.dot_general(p, p2w_ref[...], (((1,), (1,)), ((), ())),
                        preferred_element_type=jnp.float32) + p2b_ref[...]
    nrm = jnp.sqrt(jnp.sum(o * o, axis=1, keepdims=True))
    o_ref[...] = o / jnp.maximum(nrm, 1e-12)


def kernel(x, edge_index, edge_attr, batch, roi_scaler, W1, b1, W2, b2,
           P1w, P1b, gamma, beta, P2w, P2b):
    f32 = jnp.float32
    src2 = edge_index[0].reshape(ER, CH)
    dst2 = edge_index[1].reshape(ER, CH)
    w2 = edge_attr.reshape(ER, CH)
    z1 = jnp.zeros((NPS,), f32)
    z16 = jnp.zeros((NPS, 16), f32)

    dp = _k1_deg(dst2, w2, z1)

    xp = jnp.pad(x, ((0, 0), (0, 13))).reshape(226, 128, 16)
    scp = jnp.pad(jnp.tile(roi_scaler, (NUM_GRAPHS, 1)),
                  ((0, 0), (0, 13))).reshape(226, 128, 16)
    dpt = dp.reshape(NC, 226, 128).transpose(1, 0, 2)

    dis, tbl1 = pl.pallas_call(
        _t1_body,
        grid=(226,),
        in_specs=[
            pl.BlockSpec((1, 2, 128), lambda i: (i, 0, 0)),
            pl.BlockSpec((1, 128, 16), lambda i: (i, 0, 0)),
            pl.BlockSpec((1, 128, 16), lambda i: (i, 0, 0)),
        ],
        out_specs=[
            pl.BlockSpec((1, 1, 128), lambda i: (i, 0, 0)),
            pl.BlockSpec((1, 128, 16), lambda i: (i, 0, 0)),
        ],
        out_shape=[
            jax.ShapeDtypeStruct((226, 1, 128), f32),
            jax.ShapeDtypeStruct((226, 128, 16), f32),
        ],
    )(dpt, xp, scp)

    tbl1f = tbl1.reshape(N, 16)
    agg1 = _k2_conv1(src2, dst2, w2, tbl1f, z16)

    disb = dis.reshape(16, NPS, 1)
    w1p = jnp.pad(W1, ((0, 0), (0, 13)))
    gt, g0, g1 = pl.pallas_call(
        _t2_body,
        grid=(16,),
        in_specs=[
            pl.BlockSpec((1, NPS, 16), lambda i: (i, 0, 0)),
            pl.BlockSpec((1, NPS, 16), lambda i: (i, 0, 0)),
            pl.BlockSpec((1, NPS, 16), lambda i: (i, 0, 0)),
            pl.BlockSpec((1, NPS, 1), lambda i: (i, 0, 0)),
            pl.BlockSpec((128, 16), lambda i: (0, 0)),
            pl.BlockSpec((1, 128), lambda i: (0, 0)),
        ],
        out_specs=[
            pl.BlockSpec((1, NPS, 128), lambda i: (i, 0, 0)),
            pl.BlockSpec((1, NPS, 64), lambda i: (i, 0, 0)),
            pl.BlockSpec((1, NPS, 64), lambda i: (i, 0, 0)),
        ],
        out_shape=[
            jax.ShapeDtypeStruct((16, NPS, 128), f32),
            jax.ShapeDtypeStruct((16, NPS, 64), f32),
            jax.ShapeDtypeStruct((16, NPS, 64), f32),
        ],
    )(agg1[0].reshape(16, NPS, 16), agg1[1].reshape(16, NPS, 16),
      tbl1f.reshape(16, NPS, 16), disb, w1p, b1.reshape(1, 128))

    tbl2 = gt.reshape(2 * N, 64)
    agg2 = _k3_conv2(src2, dst2, w2, tbl2,
                     g0.reshape(N, 64), g1.reshape(N, 64))

    z = pl.pallas_call(
        _t3_body,
        grid=(NUM_GRAPHS,),
        in_specs=[
            pl.BlockSpec((1, NPG, 64), lambda i: (i, 0, 0)),
            pl.BlockSpec((1, NPG, 64), lambda i: (i, 0, 0)),
            pl.BlockSpec((1, NPG, 1), lambda i: (i, 0, 0)),
            pl.BlockSpec((128, 64), lambda i: (0, 0)),
            pl.BlockSpec((128, 64), lambda i: (0, 0)),
            pl.BlockSpec((1, 128), lambda i: (0, 0)),
        ],
        out_specs=pl.BlockSpec((1, 1, 128), lambda i: (i, 0, 0)),
        out_shape=jax.ShapeDtypeStruct((NUM_GRAPHS, 1, 128), f32),
    )(agg2[0].reshape(NUM_GRAPHS, NPG, 64),
      agg2[1].reshape(NUM_GRAPHS, NPG, 64),
      dis.reshape(NUM_GRAPHS, NPG, 1),
      W2[:, :64], W2[:, 64:], b2.reshape(1, 128))

    out = pl.pallas_call(
        _t4_body,
        out_shape=jax.ShapeDtypeStruct((NUM_GRAPHS, 1024), f32),
    )(z.reshape(NUM_GRAPHS, 128), P1w, P1b.reshape(1, 512),
      gamma.reshape(1, 512), beta.reshape(1, 512), P2w, P2b.reshape(1, 1024))
    return out


# speedup vs baseline: 4.6389x; 4.6389x over previous
import functools

import jax
import jax.numpy as jnp
from jax import lax
from jax.experimental import pallas as pl
from jax.experimental.pallas import tpu as pltpu, tpu_sc as plsc

NUM_GRAPHS = 256
NPG = 113
N = NUM_GRAPHS * NPG
E = N * 16
NC, NS = 2, 16
NW = NC * NS
CH = 128
ER = E // CH
RPT32 = ER // NW
RPT16 = ER // NS
NPS = N // NS

_mesh = plsc.VectorSubcoreMesh(core_axis_name="c", subcore_axis_name="s")


def _lrelu(x):
    return jnp.where(x > 0, x, 0.2 * x)


@functools.partial(
    pl.kernel,
    out_type=jax.ShapeDtypeStruct((NC, N), jnp.float32),
    mesh=_mesh,
    compiler_params=pltpu.CompilerParams(use_tc_tiling_on_sc=False),
    scratch_types=[
        pltpu.VMEM((RPT32, CH), jnp.int32),
        pltpu.VMEM((RPT32, CH), jnp.float32),
        pltpu.VMEM_SHARED((N,), jnp.float32),
    ],
)
def _k1_deg(dst_hbm, w_hbm, z1_hbm, out_hbm, dstb, wb, acc):
    c = lax.axis_index("c")
    s = lax.axis_index("s")
    wid = c * NS + s
    for t in range(15):
        j = s + NS * t

        @pl.when(j < N // 128)
        def _():
            pltpu.sync_copy(z1_hbm, acc.at[pl.ds(j * 128, 128)])

    plsc.subcore_barrier()
    pltpu.sync_copy(dst_hbm.at[wid], dstb)
    pltpu.sync_copy(w_hbm.at[wid], wb)

    def chunk(j, _):
        pltpu.sync_copy(wb.at[j], acc.at[dstb.at[j]], add=True)
        return ()

    lax.fori_loop(0, RPT32, chunk, ())
    plsc.subcore_barrier()
    for t in range(15):
        j = s + NS * t

        @pl.when(j < N // 128)
        def _():
            pltpu.sync_copy(acc.at[pl.ds(j * 128, 128)],
                            out_hbm.at[c].at[pl.ds(j * 128, 128)])


@functools.partial(
    pl.kernel,
    out_type=jax.ShapeDtypeStruct((NC, N, 64), jnp.float32),
    mesh=_mesh,
    compiler_params=pltpu.CompilerParams(use_tc_tiling_on_sc=False),
    scratch_types=[
        pltpu.VMEM((1, CH), jnp.int32),
        pltpu.VMEM((1, CH), jnp.int32),
        pltpu.VMEM((1, CH), jnp.float32),
        pltpu.VMEM((CH, 64), jnp.float32),
        pltpu.VMEM_SHARED((N, 64), jnp.float32),
        pltpu.SemaphoreType.DMA,
    ],
)
def _ka_agg(src_hbm, dst_hbm, w_hbm, tbl_hbm, g0_hbm, g1_hbm, out_hbm,
            srcb, dstb, wb, rows, acc, sem):
    c = lax.axis_index("c")
    s = lax.axis_index("s")

    @pl.when(c == 0)
    def _():
        pltpu.sync_copy(g0_hbm.at[pl.ds(s * NPS, NPS)],
                        acc.at[pl.ds(s * NPS, NPS)])

    @pl.when(c == 1)
    def _():
        pltpu.sync_copy(g1_hbm.at[pl.ds(s * NPS, NPS)],
                        acc.at[pl.ds(s * NPS, NPS)])

    plsc.subcore_barrier()

    def chunk(j, _):
        jj = s * RPT16 + j
        pltpu.sync_copy(src_hbm.at[pl.ds(jj, 1)], srcb)
        pltpu.sync_copy(dst_hbm.at[pl.ds(jj, 1)], dstb)
        pltpu.sync_copy(w_hbm.at[pl.ds(jj, 1)], wb)
        for b in range(8):
            v = srcb[0, pl.ds(b * 16, 16)]
            srcb[0, pl.ds(b * 16, 16)] = v * 2 + c
        pltpu.async_copy(tbl_hbm.at[srcb.at[0]], rows, sem).wait()

        def scale(g, _):
            wv = wb[0, pl.ds(g * 16, 16)]
            for l in range(16):
                e = g * 16 + l
                for q in range(4):
                    rows[e, pl.ds(q * 16, 16)] = (
                        rows[e, pl.ds(q * 16, 16)] * wv[l])
            return ()

        lax.fori_loop(0, CH // 16, scale, ())
        pltpu.sync_copy(rows, acc.at[dstb.at[0]], add=True)
        return ()

    lax.fori_loop(0, RPT16, chunk, ())
    plsc.subcore_barrier()
    pltpu.sync_copy(acc.at[pl.ds(s * NPS, NPS)],
                    out_hbm.at[c].at[pl.ds(s * NPS, NPS)])


def _t1_body(d0_ref, d1_ref, xp_ref, sc_ref, w1_ref,
             dis_ref, tbl_ref, t0_ref, t1_ref):
    d = d0_ref[0] + d1_ref[0] + 1.0
    dis = 1.0 / jnp.sqrt(d)
    dis_ref[0] = dis
    xs = xp_ref[0] * sc_ref[0]
    h = lax.dot_general(xs, w1_ref[...], (((1,), (1,)), ((), ())),
                        preferred_element_type=jnp.float32)
    t = h * dis
    tbl_ref[0] = t
    t0_ref[0] = t[:, :64]
    t1_ref[0] = t[:, 64:]


def _t2_body(a0_ref, a1_ref, dis_ref, b1_ref, w2_ref,
             tbl_ref, t0_ref, t1_ref):
    a = jnp.concatenate([a0_ref[0], a1_ref[0]], axis=1)
    h1 = _lrelu(a * dis_ref[0] + b1_ref[...])
    hh = lax.dot_general(h1, w2_ref[...], (((1,), (1,)), ((), ())),
                         preferred_element_type=jnp.float32)
    t = hh * dis_ref[0]
    tbl_ref[0] = t
    t0_ref[0] = t[:, :64]
    t1_ref[0] = t[:, 64:]


def _t3_body(a0_ref, a1_ref, dis_ref, b2_ref, z_ref):
    a = jnp.concatenate([a0_ref[0], a1_ref[0]], axis=1)
    h2 = _lrelu(a * dis_ref[0] + b2_ref[...])
    z_ref[0, 0, :] = jnp.max(h2, axis=0)


def _t4_body(z_ref, p1w_ref, p1b_ref, g_ref, b_ref, p2w_ref, p2b_ref, o_ref):
    p = lax.dot_general(z_ref[...], p1w_ref[...], (((1,), (1,)), ((), ())),
                        preferred_element_type=jnp.float32) + p1b_ref[...]
    mean = jnp.mean(p, axis=0, keepdims=True)
    var = jnp.mean((p - mean) ** 2, axis=0, keepdims=True)
    p = (p - mean) / jnp.sqrt(var + 1e-5) * g_ref[...] + b_ref[...]
    p = jnp.maximum(p, 0.0)
    o = lax.dot_general(p, p2w_ref[...], (((1,), (1,)), ((), ())),
                        preferred_element_type=jnp.float32) + p2b_ref[...]
    nrm = jnp.sqrt(jnp.sum(o * o, axis=1, keepdims=True))
    o_ref[...] = o / jnp.maximum(nrm, 1e-12)


def kernel(x, edge_index, edge_attr, batch, roi_scaler, W1, b1, W2, b2,
           P1w, P1b, gamma, beta, P2w, P2b):
    f32 = jnp.float32
    src32 = edge_index[0].reshape(NW, RPT32, CH)
    dst32 = edge_index[1].reshape(NW, RPT32, CH)
    w32 = edge_attr.reshape(NW, RPT32, CH)
    src2 = edge_index[0].reshape(ER, CH)
    dst2 = edge_index[1].reshape(ER, CH)
    w2 = edge_attr.reshape(ER, CH)
    z1 = jnp.zeros((128,), f32)

    dp = _k1_deg(dst32, w32, z1)

    xp = jnp.pad(x, ((0, 0), (0, 13))).reshape(NS, NPS, 16)
    scp = jnp.pad(jnp.tile(roi_scaler, (NUM_GRAPHS, 1)),
                  ((0, 0), (0, 13))).reshape(NS, NPS, 16)
    w1p = jnp.pad(W1, ((0, 0), (0, 13)))

    blk_nps1 = pl.BlockSpec((1, NPS, 1), lambda i: (i, 0, 0))
    blk_nps16 = pl.BlockSpec((1, NPS, 16), lambda i: (i, 0, 0))
    blk_nps64 = pl.BlockSpec((1, NPS, 64), lambda i: (i, 0, 0))
    blk_nps128 = pl.BlockSpec((1, NPS, 128), lambda i: (i, 0, 0))

    dis, tblA, ta0, ta1 = pl.pallas_call(
        _t1_body,
        grid=(NS,),
        in_specs=[
            blk_nps1, blk_nps1, blk_nps16, blk_nps16,
            pl.BlockSpec((128, 16), lambda i: (0, 0)),
        ],
        out_specs=[blk_nps1, blk_nps128, blk_nps64, blk_nps64],
        out_shape=[
            jax.ShapeDtypeStruct((NS, NPS, 1), f32),
            jax.ShapeDtypeStruct((NS, NPS, 128), f32),
            jax.ShapeDtypeStruct((NS, NPS, 64), f32),
            jax.ShapeDtypeStruct((NS, NPS, 64), f32),
        ],
    )(dp[0].reshape(NS, NPS, 1), dp[1].reshape(NS, NPS, 1), xp, scp, w1p)

    agg1 = _ka_agg(src2, dst2, w2, tblA.reshape(2 * N, 64),
                   ta0.reshape(N, 64), ta1.reshape(N, 64))

    tblB, tb0, tb1 = pl.pallas_call(
        _t2_body,
        grid=(NS,),
        in_specs=[
            blk_nps64, blk_nps64, blk_nps1,
            pl.BlockSpec((1, 128), lambda i: (0, 0)),
            pl.BlockSpec((128, 128), lambda i: (0, 0)),
        ],
        out_specs=[blk_nps128, blk_nps64, blk_nps64],
        out_shape=[
            jax.ShapeDtypeStruct((NS, NPS, 128), f32),
            jax.ShapeDtypeStruct((NS, NPS, 64), f32),
            jax.ShapeDtypeStruct((NS, NPS, 64), f32),
        ],
    )(agg1[0].reshape(NS, NPS, 64), agg1[1].reshape(NS, NPS, 64),
      dis.reshape(NS, NPS, 1), b1.reshape(1, 128), W2)

    agg2 = _ka_agg(src2, dst2, w2, tblB.reshape(2 * N, 64),
                   tb0.reshape(N, 64), tb1.reshape(N, 64))

    z = pl.pallas_call(
        _t3_body,
        grid=(NUM_GRAPHS,),
        in_specs=[
            pl.BlockSpec((1, NPG, 64), lambda i: (i, 0, 0)),
            pl.BlockSpec((1, NPG, 64), lambda i: (i, 0, 0)),
            pl.BlockSpec((1, NPG, 1), lambda i: (i, 0, 0)),
            pl.BlockSpec((1, 128), lambda i: (0, 0)),
        ],
        out_specs=pl.BlockSpec((1, 1, 128), lambda i: (i, 0, 0)),
        out_shape=jax.ShapeDtypeStruct((NUM_GRAPHS, 1, 128), f32),
    )(agg2[0].reshape(NUM_GRAPHS, NPG, 64),
      agg2[1].reshape(NUM_GRAPHS, NPG, 64),
      dis.reshape(NUM_GRAPHS, NPG, 1), b2.reshape(1, 128))

    out = pl.pallas_call(
        _t4_body,
        out_shape=jax.ShapeDtypeStruct((NUM_GRAPHS, 1024), f32),
    )(z.reshape(NUM_GRAPHS, 128), P1w, P1b.reshape(1, 512),
      gamma.reshape(1, 512), beta.reshape(1, 512), P2w, P2b.reshape(1, 1024))
    return out


# speedup vs baseline: 5.8017x; 1.2507x over previous
import functools

import jax
import jax.numpy as jnp
from jax import lax
from jax.experimental import pallas as pl
from jax.experimental.pallas import tpu as pltpu, tpu_sc as plsc

NUM_GRAPHS = 256
NPG = 113
N = NUM_GRAPHS * NPG
E = N * 16
NC, NS = 2, 16
NW = NC * NS
CH = 128
ER = E // CH
RPT32 = ER // NW
RPT16 = ER // NS
NPS = N // NS

_mesh = plsc.VectorSubcoreMesh(core_axis_name="c", subcore_axis_name="s")


def _lrelu(x):
    return jnp.where(x > 0, x, 0.2 * x)


@functools.partial(
    pl.kernel,
    out_type=jax.ShapeDtypeStruct((NC, N), jnp.float32),
    mesh=_mesh,
    compiler_params=pltpu.CompilerParams(use_tc_tiling_on_sc=False),
    scratch_types=[
        pltpu.VMEM((RPT32, CH), jnp.int32),
        pltpu.VMEM((RPT32, CH), jnp.float32),
        pltpu.VMEM_SHARED((N,), jnp.float32),
    ],
)
def _k1_deg(dst_hbm, w_hbm, z1_hbm, out_hbm, dstb, wb, acc):
    c = lax.axis_index("c")
    s = lax.axis_index("s")
    wid = c * NS + s
    for t in range(15):
        j = s + NS * t

        @pl.when(j < N // 128)
        def _():
            pltpu.sync_copy(z1_hbm, acc.at[pl.ds(j * 128, 128)])

    plsc.subcore_barrier()
    pltpu.sync_copy(dst_hbm.at[wid], dstb)
    pltpu.sync_copy(w_hbm.at[wid], wb)

    def chunk(j, _):
        pltpu.sync_copy(wb.at[j], acc.at[dstb.at[j]], add=True)
        return ()

    lax.fori_loop(0, RPT32, chunk, ())
    plsc.subcore_barrier()
    for t in range(15):
        j = s + NS * t

        @pl.when(j < N // 128)
        def _():
            pltpu.sync_copy(acc.at[pl.ds(j * 128, 128)],
                            out_hbm.at[c].at[pl.ds(j * 128, 128)])


EC64 = E // 64
CPT = EC64 // NS
QPT = CPT // 4


@functools.partial(
    pl.kernel,
    out_type=jax.ShapeDtypeStruct((NC, N, 64), jnp.float32),
    mesh=_mesh,
    compiler_params=pltpu.CompilerParams(use_tc_tiling_on_sc=False,
                                         needs_layout_passes=False),
    scratch_types=[
        pltpu.VMEM((2, 3, 64), jnp.int32),
        pltpu.VMEM((2, 3, 64), jnp.int32),
        pltpu.VMEM((64, 64), jnp.float32),
        pltpu.VMEM((64, 64), jnp.float32),
        pltpu.VMEM_SHARED((N, 64), jnp.float32),
        pltpu.SemaphoreType.DMA,
        pltpu.SemaphoreType.DMA,
        pltpu.SemaphoreType.DMA,
        pltpu.SemaphoreType.DMA,
    ],
)
def _ka_agg(epk_hbm, tbl_hbm, gini_hbm, out_hbm,
            eba, ebb, rows0, rows1, acc, gs0, gs1, ss0, ss1):
    c = lax.axis_index("c")
    s = lax.axis_index("s")
    cn = c * N
    base = s * CPT

    pltpu.sync_copy(gini_hbm.at[c].at[pl.ds(s * NPS, NPS)],
                    acc.at[pl.ds(s * NPS, NPS)])
    plsc.subcore_barrier()

    def _stage(eb, row):
        pltpu.sync_copy(epk_hbm.at[pl.ds(row, 2)], eb)
        for r in range(2):
            for g in range(4):
                v = eb[r, 0, pl.ds(g * 16, 16)]
                eb[r, 0, pl.ds(g * 16, 16)] = v + cn

    def _scale(rows, eb, r):
        def grp(g, _):
            wv = plsc.bitcast(eb[r, 2, pl.ds(g * 16, 16)], jnp.float32)
            for l in range(16):
                e = g * 16 + l
                for q in range(4):
                    rows[e, pl.ds(q * 16, 16)] = (
                        rows[e, pl.ds(q * 16, 16)] * wv[l])
            return ()

        lax.fori_loop(0, 4, grp, ())

    def _gather(eb, r, rows, sem):
        pltpu.async_copy(tbl_hbm.at[eb.at[r, 0]], rows, sem)

    def _gwait(eb, r, rows, sem):
        pltpu.make_async_copy(tbl_hbm.at[eb.at[r, 0]], rows, sem).wait()

    def _scatter(rows, eb, r, sem):
        pltpu.async_copy(rows, acc.at[eb.at[r, 1]], sem, add=True)

    def _swait(rows, eb, r, sem):
        pltpu.make_async_copy(rows, acc.at[eb.at[r, 1]], sem).wait()

    _stage(eba, base)
    _gather(eba, 0, rows0, gs0)
    _gather(eba, 1, rows1, gs1)

    def quad(k, _):
        q0 = base + 4 * k
        _gwait(eba, 0, rows0, gs0)
        _scale(rows0, eba, 0)
        _scatter(rows0, eba, 0, ss0)
        _gwait(eba, 1, rows1, gs1)
        _scale(rows1, eba, 1)
        _scatter(rows1, eba, 1, ss1)
        _stage(ebb, q0 + 2)
        _swait(rows0, eba, 0, ss0)
        _gather(ebb, 0, rows0, gs0)
        _swait(rows1, eba, 1, ss1)
        _gather(ebb, 1, rows1, gs1)
        _gwait(ebb, 0, rows0, gs0)
        _scale(rows0, ebb, 0)
        _scatter(rows0, ebb, 0, ss0)
        _gwait(ebb, 1, rows1, gs1)
        _scale(rows1, ebb, 1)
        _scatter(rows1, ebb, 1, ss1)

        @pl.when(k < QPT - 1)
        def _():
            _stage(eba, q0 + 4)
            _swait(rows0, ebb, 0, ss0)
            _gather(eba, 0, rows0, gs0)
            _swait(rows1, ebb, 1, ss1)
            _gather(eba, 1, rows1, gs1)

        return ()

    lax.fori_loop(0, QPT, quad, ())
    _swait(rows0, ebb, 0, ss0)
    _swait(rows1, ebb, 1, ss1)
    plsc.subcore_barrier()
    pltpu.sync_copy(acc.at[pl.ds(s * NPS, NPS)],
                    out_hbm.at[c].at[pl.ds(s * NPS, NPS)])


def _t1_body(d0_ref, d1_ref, xp_ref, sc_ref, w1_ref, dis_ref, tbl_ref):
    d = d0_ref[0] + d1_ref[0] + 1.0
    dis = 1.0 / jnp.sqrt(d)
    dis_ref[0] = dis
    xs = xp_ref[0] * sc_ref[0]
    h = lax.dot_general(xs, w1_ref[...], (((1,), (1,)), ((), ())),
                        preferred_element_type=jnp.float32)
    t = h * dis
    tbl_ref[0, 0] = t[:, :64]
    tbl_ref[1, 0] = t[:, 64:]


def _t2_body(a0_ref, a1_ref, dis_ref, b1_ref, w2_ref, tbl_ref):
    a = jnp.concatenate([a0_ref[0], a1_ref[0]], axis=1)
    h1 = _lrelu(a * dis_ref[0] + b1_ref[...])
    hh = lax.dot_general(h1, w2_ref[...], (((1,), (1,)), ((), ())),
                         preferred_element_type=jnp.float32)
    t = hh * dis_ref[0]
    tbl_ref[0, 0] = t[:, :64]
    tbl_ref[1, 0] = t[:, 64:]


def _t3_body(a0_ref, a1_ref, dis_ref, b2_ref, z_ref):
    a = jnp.concatenate([a0_ref[0], a1_ref[0]], axis=1)
    h2 = _lrelu(a * dis_ref[0] + b2_ref[...])
    z_ref[0, 0, :] = jnp.max(h2, axis=0)


def _t4_body(z_ref, p1w_ref, p1b_ref, g_ref, b_ref, p2w_ref, p2b_ref, o_ref):
    p = lax.dot_general(z_ref[...], p1w_ref[...], (((1,), (1,)), ((), ())),
                        preferred_element_type=jnp.float32) + p1b_ref[...]
    mean = jnp.mean(p, axis=0, keepdims=True)
    var = jnp.mean((p - mean) ** 2, axis=0, keepdims=True)
    p = (p - mean) / jnp.sqrt(var + 1e-5) * g_ref[...] + b_ref[...]
    p = jnp.maximum(p, 0.0)
    o = lax.dot_general(p, p2w_ref[...], (((1,), (1,)), ((), ())),
                        preferred_element_type=jnp.float32) + p2b_ref[...]
    nrm = jnp.sqrt(jnp.sum(o * o, axis=1, keepdims=True))
    o_ref[...] = o / jnp.maximum(nrm, 1e-12)


def kernel(x, edge_index, edge_attr, batch, roi_scaler, W1, b1, W2, b2,
           P1w, P1b, gamma, beta, P2w, P2b):
    f32 = jnp.float32
    src32 = edge_index[0].reshape(NW, RPT32, CH)
    dst32 = edge_index[1].reshape(NW, RPT32, CH)
    w32 = edge_attr.reshape(NW, RPT32, CH)
    z1 = jnp.zeros((128,), f32)
    wbits = lax.bitcast_convert_type(edge_attr, jnp.int32)
    epk = jnp.concatenate([
        edge_index[0].reshape(EC64, 1, 64),
        edge_index[1].reshape(EC64, 1, 64),
        wbits.reshape(EC64, 1, 64)], axis=1)

    dp = _k1_deg(dst32, w32, z1)

    xp = jnp.pad(x, ((0, 0), (0, 13))).reshape(NS, NPS, 16)
    scp = jnp.pad(jnp.tile(roi_scaler, (NUM_GRAPHS, 1)),
                  ((0, 0), (0, 13))).reshape(NS, NPS, 16)
    w1p = jnp.pad(W1, ((0, 0), (0, 13)))

    blk_nps1 = pl.BlockSpec((1, NPS, 1), lambda i: (i, 0, 0))
    blk_nps16 = pl.BlockSpec((1, NPS, 16), lambda i: (i, 0, 0))
    blk_nps64 = pl.BlockSpec((1, NPS, 64), lambda i: (i, 0, 0))

    blk_tbl = pl.BlockSpec((2, 1, NPS, 64), lambda i: (0, i, 0, 0))
    dis, tblA = pl.pallas_call(
        _t1_body,
        grid=(NS,),
        in_specs=[
            blk_nps1, blk_nps1, blk_nps16, blk_nps16,
            pl.BlockSpec((128, 16), lambda i: (0, 0)),
        ],
        out_specs=[blk_nps1, blk_tbl],
        out_shape=[
            jax.ShapeDtypeStruct((NS, NPS, 1), f32),
            jax.ShapeDtypeStruct((2, NS, NPS, 64), f32),
        ],
    )(dp[0].reshape(NS, NPS, 1), dp[1].reshape(NS, NPS, 1), xp, scp, w1p)

    agg1 = _ka_agg(epk, tblA.reshape(2 * N, 64),
                   tblA.reshape(2, N, 64))

    tblB = pl.pallas_call(
        _t2_body,
        grid=(NS,),
        in_specs=[
            blk_nps64, blk_nps64, blk_nps1,
            pl.BlockSpec((1, 128), lambda i: (0, 0)),
            pl.BlockSpec((128, 128), lambda i: (0, 0)),
        ],
        out_specs=blk_tbl,
        out_shape=jax.ShapeDtypeStruct((2, NS, NPS, 64), f32),
    )(agg1[0].reshape(NS, NPS, 64), agg1[1].reshape(NS, NPS, 64),
      dis.reshape(NS, NPS, 1), b1.reshape(1, 128), W2)

    agg2 = _ka_agg(epk, tblB.reshape(2 * N, 64),
                   tblB.reshape(2, N, 64))

    z = pl.pallas_call(
        _t3_body,
        grid=(NUM_GRAPHS,),
        in_specs=[
            pl.BlockSpec((1, NPG, 64), lambda i: (i, 0, 0)),
            pl.BlockSpec((1, NPG, 64), lambda i: (i, 0, 0)),
            pl.BlockSpec((1, NPG, 1), lambda i: (i, 0, 0)),
            pl.BlockSpec((1, 128), lambda i: (0, 0)),
        ],
        out_specs=pl.BlockSpec((1, 1, 128), lambda i: (i, 0, 0)),
        out_shape=jax.ShapeDtypeStruct((NUM_GRAPHS, 1, 128), f32),
    )(agg2[0].reshape(NUM_GRAPHS, NPG, 64),
      agg2[1].reshape(NUM_GRAPHS, NPG, 64),
      dis.reshape(NUM_GRAPHS, NPG, 1), b2.reshape(1, 128))

    out = pl.pallas_call(
        _t4_body,
        out_shape=jax.ShapeDtypeStruct((NUM_GRAPHS, 1024), f32),
    )(z.reshape(NUM_GRAPHS, 128), P1w, P1b.reshape(1, 512),
      gamma.reshape(1, 512), beta.reshape(1, 512), P2w, P2b.reshape(1, 1024))
    return out


# speedup vs baseline: 6.1592x; 1.0616x over previous
import functools

import jax
import jax.numpy as jnp
from jax import lax
from jax.experimental import pallas as pl
from jax.experimental.pallas import tpu as pltpu, tpu_sc as plsc

NUM_GRAPHS = 256
NPG = 113
N = NUM_GRAPHS * NPG
E = N * 16
NC, NS = 2, 16
NW = NC * NS
CH = 128
ER = E // CH
RPT32 = ER // NW
RPT16 = ER // NS
NPS = N // NS

_mesh = plsc.VectorSubcoreMesh(core_axis_name="c", subcore_axis_name="s")


def _lrelu(x):
    return jnp.where(x > 0, x, 0.2 * x)


@functools.partial(
    pl.kernel,
    out_type=jax.ShapeDtypeStruct((NC, N), jnp.float32),
    mesh=_mesh,
    compiler_params=pltpu.CompilerParams(use_tc_tiling_on_sc=False),
    scratch_types=[
        pltpu.VMEM((RPT32, CH), jnp.int32),
        pltpu.VMEM((RPT32, CH), jnp.float32),
        pltpu.VMEM_SHARED((N,), jnp.float32),
        pltpu.SemaphoreType.DMA,
    ],
)
def _k1_deg(dst_hbm, w_hbm, z1_hbm, out_hbm, dstb, wb, acc, sem1):
    c = lax.axis_index("c")
    s = lax.axis_index("s")
    wid = c * NS + s
    for t in range(15):
        j = s + NS * t

        @pl.when(j < N // 128)
        def _():
            pltpu.sync_copy(z1_hbm, acc.at[pl.ds(j * 128, 128)])

    plsc.subcore_barrier()
    pltpu.sync_copy(dst_hbm.at[wid], dstb)
    pltpu.sync_copy(w_hbm.at[wid], wb)

    def chunk(j, _):
        for u in range(8):
            pltpu.async_copy(wb.at[j * 8 + u], acc.at[dstb.at[j * 8 + u]],
                             sem1, add=True)
        for u in range(8):
            pltpu.make_async_copy(wb.at[j * 8 + u],
                                  acc.at[dstb.at[j * 8 + u]], sem1).wait()
        return ()

    lax.fori_loop(0, RPT32 // 8, chunk, ())
    pltpu.sync_copy(wb.at[RPT32 - 1], acc.at[dstb.at[RPT32 - 1]], add=True)
    plsc.subcore_barrier()
    for t in range(15):
        j = s + NS * t

        @pl.when(j < N // 128)
        def _():
            pltpu.sync_copy(acc.at[pl.ds(j * 128, 128)],
                            out_hbm.at[c].at[pl.ds(j * 128, 128)])


EC64 = E // 64
CPT = EC64 // NS
QPT = CPT // 4


@functools.partial(
    pl.kernel,
    out_type=jax.ShapeDtypeStruct((NC, N, 64), jnp.float32),
    mesh=_mesh,
    compiler_params=pltpu.CompilerParams(use_tc_tiling_on_sc=False,
                                         needs_layout_passes=False),
    scratch_types=[
        pltpu.VMEM((2, 3, 64), jnp.int32),
        pltpu.VMEM((2, 3, 64), jnp.int32),
        pltpu.VMEM((64, 64), jnp.float32),
        pltpu.VMEM((64, 64), jnp.float32),
        pltpu.VMEM_SHARED((N, 64), jnp.float32),
        pltpu.SemaphoreType.DMA,
        pltpu.SemaphoreType.DMA,
        pltpu.SemaphoreType.DMA,
        pltpu.SemaphoreType.DMA,
    ],
)
def _ka_agg(epk_hbm, tbl_hbm, gini_hbm, out_hbm,
            eba, ebb, rows0, rows1, acc, gs0, gs1, ss0, ss1):
    c = lax.axis_index("c")
    s = lax.axis_index("s")
    cn = c * N
    base = s * CPT

    pltpu.sync_copy(gini_hbm.at[c].at[pl.ds(s * NPS, NPS)],
                    acc.at[pl.ds(s * NPS, NPS)])
    plsc.subcore_barrier()

    def _stage(eb, row):
        pltpu.sync_copy(epk_hbm.at[pl.ds(row, 2)], eb)
        for r in range(2):
            for g in range(4):
                v = eb[r, 0, pl.ds(g * 16, 16)]
                eb[r, 0, pl.ds(g * 16, 16)] = v + cn

    def _scale(rows, eb, r):
        def grp(g, _):
            wv = plsc.bitcast(eb[r, 2, pl.ds(g * 16, 16)], jnp.float32)
            for l in range(16):
                e = g * 16 + l
                for q in range(4):
                    rows[e, pl.ds(q * 16, 16)] = (
                        rows[e, pl.ds(q * 16, 16)] * wv[l])
            return ()

        lax.fori_loop(0, 4, grp, ())

    def _gather(eb, r, rows, sem):
        pltpu.async_copy(tbl_hbm.at[eb.at[r, 0]], rows, sem)

    def _gwait(eb, r, rows, sem):
        pltpu.make_async_copy(tbl_hbm.at[eb.at[r, 0]], rows, sem).wait()

    def _scatter(rows, eb, r, sem):
        pltpu.async_copy(rows, acc.at[eb.at[r, 1]], sem, add=True)

    def _swait(rows, eb, r, sem):
        pltpu.make_async_copy(rows, acc.at[eb.at[r, 1]], sem).wait()

    _stage(eba, base)
    _gather(eba, 0, rows0, gs0)
    _gather(eba, 1, rows1, gs1)

    def quad(k, _):
        q0 = base + 4 * k
        _gwait(eba, 0, rows0, gs0)
        _scale(rows0, eba, 0)
        _scatter(rows0, eba, 0, ss0)
        _stage(ebb, q0 + 2)
        _gwait(eba, 1, rows1, gs1)
        _scale(rows1, eba, 1)
        _scatter(rows1, eba, 1, ss1)
        _swait(rows0, eba, 0, ss0)
        _gather(ebb, 0, rows0, gs0)
        _swait(rows1, eba, 1, ss1)
        _gather(ebb, 1, rows1, gs1)

        @pl.when(k < QPT - 1)
        def _():
            _stage(eba, q0 + 4)

        _gwait(ebb, 0, rows0, gs0)
        _scale(rows0, ebb, 0)
        _scatter(rows0, ebb, 0, ss0)
        _gwait(ebb, 1, rows1, gs1)
        _scale(rows1, ebb, 1)
        _scatter(rows1, ebb, 1, ss1)

        @pl.when(k < QPT - 1)
        def _():
            _swait(rows0, ebb, 0, ss0)
            _gather(eba, 0, rows0, gs0)
            _swait(rows1, ebb, 1, ss1)
            _gather(eba, 1, rows1, gs1)

        return ()

    lax.fori_loop(0, QPT, quad, ())
    _swait(rows0, ebb, 0, ss0)
    _swait(rows1, ebb, 1, ss1)
    plsc.subcore_barrier()
    pltpu.sync_copy(acc.at[pl.ds(s * NPS, NPS)],
                    out_hbm.at[c].at[pl.ds(s * NPS, NPS)])


def _t1_body(d0_ref, d1_ref, xp_ref, sc_ref, w1_ref, dis_ref, tbl_ref):
    d = d0_ref[0] + d1_ref[0] + 1.0
    dis = 1.0 / jnp.sqrt(d)
    dis_ref[0] = dis
    xs = xp_ref[0] * sc_ref[0]
    h = lax.dot_general(xs, w1_ref[...], (((1,), (1,)), ((), ())),
                        preferred_element_type=jnp.float32)
    t = h * dis
    tbl_ref[0, 0] = t[:, :64]
    tbl_ref[1, 0] = t[:, 64:]


def _t2_body(a0_ref, a1_ref, dis_ref, b1_ref, w2_ref, tbl_ref):
    a = jnp.concatenate([a0_ref[0], a1_ref[0]], axis=1)
    h1 = _lrelu(a * dis_ref[0] + b1_ref[...])
    hh = lax.dot_general(h1, w2_ref[...], (((1,), (1,)), ((), ())),
                         preferred_element_type=jnp.float32)
    t = hh * dis_ref[0]
    tbl_ref[0, 0] = t[:, :64]
    tbl_ref[1, 0] = t[:, 64:]


def _t3_body(a0_ref, a1_ref, dis_ref, b2_ref, z_ref):
    a = jnp.concatenate([a0_ref[0], a1_ref[0]], axis=1)
    h2 = _lrelu(a * dis_ref[0] + b2_ref[...])
    z_ref[0, 0, :] = jnp.max(h2, axis=0)


def _t4_body(z_ref, p1w_ref, p1b_ref, g_ref, b_ref, p2w_ref, p2b_ref, o_ref):
    p = lax.dot_general(z_ref[...], p1w_ref[...], (((1,), (1,)), ((), ())),
                        preferred_element_type=jnp.float32) + p1b_ref[...]
    mean = jnp.mean(p, axis=0, keepdims=True)
    var = jnp.mean((p - mean) ** 2, axis=0, keepdims=True)
    p = (p - mean) / jnp.sqrt(var + 1e-5) * g_ref[...] + b_ref[...]
    p = jnp.maximum(p, 0.0)
    o = lax.dot_general(p, p2w_ref[...], (((1,), (1,)), ((), ())),
                        preferred_element_type=jnp.float32) + p2b_ref[...]
    nrm = jnp.sqrt(jnp.sum(o * o, axis=1, keepdims=True))
    o_ref[...] = o / jnp.maximum(nrm, 1e-12)


def kernel(x, edge_index, edge_attr, batch, roi_scaler, W1, b1, W2, b2,
           P1w, P1b, gamma, beta, P2w, P2b):
    f32 = jnp.float32
    src32 = edge_index[0].reshape(NW, RPT32, CH)
    dst32 = edge_index[1].reshape(NW, RPT32, CH)
    w32 = edge_attr.reshape(NW, RPT32, CH)
    z1 = jnp.zeros((128,), f32)
    wbits = lax.bitcast_convert_type(edge_attr, jnp.int32)
    epk = jnp.concatenate([
        edge_index[0].reshape(EC64, 1, 64),
        edge_index[1].reshape(EC64, 1, 64),
        wbits.reshape(EC64, 1, 64)], axis=1)

    dp = _k1_deg(dst32, w32, z1)

    xp = jnp.pad(x, ((0, 0), (0, 13))).reshape(NS, NPS, 16)
    scp = jnp.pad(jnp.tile(roi_scaler, (NUM_GRAPHS, 1)),
                  ((0, 0), (0, 13))).reshape(NS, NPS, 16)
    w1p = jnp.pad(W1, ((0, 0), (0, 13)))

    blk_nps1 = pl.BlockSpec((1, NPS, 1), lambda i: (i, 0, 0))
    blk_nps16 = pl.BlockSpec((1, NPS, 16), lambda i: (i, 0, 0))
    blk_nps64 = pl.BlockSpec((1, NPS, 64), lambda i: (i, 0, 0))

    blk_tbl = pl.BlockSpec((2, 1, NPS, 64), lambda i: (0, i, 0, 0))
    dis, tblA = pl.pallas_call(
        _t1_body,
        grid=(NS,),
        in_specs=[
            blk_nps1, blk_nps1, blk_nps16, blk_nps16,
            pl.BlockSpec((128, 16), lambda i: (0, 0)),
        ],
        out_specs=[blk_nps1, blk_tbl],
        out_shape=[
            jax.ShapeDtypeStruct((NS, NPS, 1), f32),
            jax.ShapeDtypeStruct((2, NS, NPS, 64), f32),
        ],
    )(dp[0].reshape(NS, NPS, 1), dp[1].reshape(NS, NPS, 1), xp, scp, w1p)

    agg1 = _ka_agg(epk, tblA.reshape(2 * N, 64),
                   tblA.reshape(2, N, 64))

    tblB = pl.pallas_call(
        _t2_body,
        grid=(NS,),
        in_specs=[
            blk_nps64, blk_nps64, blk_nps1,
            pl.BlockSpec((1, 128), lambda i: (0, 0)),
            pl.BlockSpec((128, 128), lambda i: (0, 0)),
        ],
        out_specs=blk_tbl,
        out_shape=jax.ShapeDtypeStruct((2, NS, NPS, 64), f32),
    )(agg1[0].reshape(NS, NPS, 64), agg1[1].reshape(NS, NPS, 64),
      dis.reshape(NS, NPS, 1), b1.reshape(1, 128), W2)

    agg2 = _ka_agg(epk, tblB.reshape(2 * N, 64),
                   tblB.reshape(2, N, 64))

    z = pl.pallas_call(
        _t3_body,
        grid=(NUM_GRAPHS,),
        in_specs=[
            pl.BlockSpec((1, NPG, 64), lambda i: (i, 0, 0)),
            pl.BlockSpec((1, NPG, 64), lambda i: (i, 0, 0)),
            pl.BlockSpec((1, NPG, 1), lambda i: (i, 0, 0)),
            pl.BlockSpec((1, 128), lambda i: (0, 0)),
        ],
        out_specs=pl.BlockSpec((1, 1, 128), lambda i: (i, 0, 0)),
        out_shape=jax.ShapeDtypeStruct((NUM_GRAPHS, 1, 128), f32),
    )(agg2[0].reshape(NUM_GRAPHS, NPG, 64),
      agg2[1].reshape(NUM_GRAPHS, NPG, 64),
      dis.reshape(NUM_GRAPHS, NPG, 1), b2.reshape(1, 128))

    out = pl.pallas_call(
        _t4_body,
        out_shape=jax.ShapeDtypeStruct((NUM_GRAPHS, 1024), f32),
    )(z.reshape(NUM_GRAPHS, 128), P1w, P1b.reshape(1, 512),
      gamma.reshape(1, 512), beta.reshape(1, 512), P2w, P2b.reshape(1, 1024))
    return out


# speedup vs baseline: 6.1655x; 1.0010x over previous
import functools

import jax
import jax.numpy as jnp
from jax import lax
from jax.experimental import pallas as pl
from jax.experimental.pallas import tpu as pltpu, tpu_sc as plsc

NUM_GRAPHS = 256
NPG = 113
N = NUM_GRAPHS * NPG
E = N * 16
NC, NS = 2, 16
NW = NC * NS
CH = 128
ER = E // CH
RPT32 = ER // NW
RPT16 = ER // NS
NPS = N // NS

_mesh = plsc.VectorSubcoreMesh(core_axis_name="c", subcore_axis_name="s")


def _lrelu(x):
    return jnp.where(x > 0, x, 0.2 * x)


@functools.partial(
    pl.kernel,
    out_type=jax.ShapeDtypeStruct((NC, N), jnp.float32),
    mesh=_mesh,
    compiler_params=pltpu.CompilerParams(use_tc_tiling_on_sc=False),
    scratch_types=[
        pltpu.VMEM((RPT32, CH), jnp.int32),
        pltpu.VMEM((RPT32, CH), jnp.float32),
        pltpu.VMEM_SHARED((N,), jnp.float32),
        pltpu.SemaphoreType.DMA,
    ],
)
def _k1_deg(dst_hbm, w_hbm, z1_hbm, out_hbm, dstb, wb, acc, sem1):
    c = lax.axis_index("c")
    s = lax.axis_index("s")
    wid = c * NS + s
    for t in range(15):
        j = s + NS * t

        @pl.when(j < N // 128)
        def _():
            pltpu.sync_copy(z1_hbm, acc.at[pl.ds(j * 128, 128)])

    plsc.subcore_barrier()
    pltpu.sync_copy(dst_hbm.at[wid], dstb)
    pltpu.sync_copy(w_hbm.at[wid], wb)

    def chunk(j, _):
        for u in range(8):
            pltpu.async_copy(wb.at[j * 8 + u], acc.at[dstb.at[j * 8 + u]],
                             sem1, add=True)
        for u in range(8):
            pltpu.make_async_copy(wb.at[j * 8 + u],
                                  acc.at[dstb.at[j * 8 + u]], sem1).wait()
        return ()

    lax.fori_loop(0, RPT32 // 8, chunk, ())
    pltpu.sync_copy(wb.at[RPT32 - 1], acc.at[dstb.at[RPT32 - 1]], add=True)
    plsc.subcore_barrier()
    for t in range(15):
        j = s + NS * t

        @pl.when(j < N // 128)
        def _():
            pltpu.sync_copy(acc.at[pl.ds(j * 128, 128)],
                            out_hbm.at[c].at[pl.ds(j * 128, 128)])


EC64 = E // 64
CPT = EC64 // NS
QPT = CPT // 4


@functools.partial(
    pl.kernel,
    out_type=jax.ShapeDtypeStruct((NC, N, 64), jnp.float32),
    mesh=_mesh,
    compiler_params=pltpu.CompilerParams(use_tc_tiling_on_sc=False,
                                         needs_layout_passes=False),
    scratch_types=[
        pltpu.VMEM((2, 3, 64), jnp.int32),
        pltpu.VMEM((2, 3, 64), jnp.int32),
        pltpu.VMEM((64, 64), jnp.float32),
        pltpu.VMEM((64, 64), jnp.float32),
        pltpu.VMEM_SHARED((N, 64), jnp.float32),
        pltpu.SemaphoreType.DMA,
        pltpu.SemaphoreType.DMA,
        pltpu.SemaphoreType.DMA,
        pltpu.SemaphoreType.DMA,
    ],
)
def _ka_agg(epk_hbm, tbl_hbm, gini_hbm, out_hbm,
            eba, ebb, rows0, rows1, acc, gs0, gs1, ss0, ss1):
    c = lax.axis_index("c")
    s = lax.axis_index("s")
    cn = c * N
    base = s * CPT

    pltpu.sync_copy(gini_hbm.at[c].at[pl.ds(s * NPS, NPS)],
                    acc.at[pl.ds(s * NPS, NPS)])
    plsc.subcore_barrier()

    def _stage(eb, row):
        pltpu.sync_copy(epk_hbm.at[pl.ds(row, 2)], eb)
        for r in range(2):
            for g in range(4):
                v = eb[r, 0, pl.ds(g * 16, 16)]
                eb[r, 0, pl.ds(g * 16, 16)] = v + cn

    def _scale(rows, eb, r):
        def grp(g, _):
            wv = plsc.bitcast(eb[r, 2, pl.ds(g * 16, 16)], jnp.float32)
            for l in range(16):
                e = g * 16 + l
                for q in range(4):
                    rows[e, pl.ds(q * 16, 16)] = (
                        rows[e, pl.ds(q * 16, 16)] * wv[l])
            return ()

        lax.fori_loop(0, 4, grp, ())

    def _gather(eb, r, rows, sem):
        pltpu.async_copy(tbl_hbm.at[eb.at[r, 0]], rows, sem)

    def _gwait(eb, r, rows, sem):
        pltpu.make_async_copy(tbl_hbm.at[eb.at[r, 0]], rows, sem).wait()

    def _scatter(rows, eb, r, sem):
        pltpu.async_copy(rows, acc.at[eb.at[r, 1]], sem, add=True)

    def _swait(rows, eb, r, sem):
        pltpu.make_async_copy(rows, acc.at[eb.at[r, 1]], sem).wait()

    _stage(eba, base)
    _gather(eba, 0, rows0, gs0)
    _gather(eba, 1, rows1, gs1)

    def quad(k, _):
        q0 = base + 4 * k
        _gwait(eba, 0, rows0, gs0)
        _scale(rows0, eba, 0)
        _scatter(rows0, eba, 0, ss0)
        _stage(ebb, q0 + 2)
        _gwait(eba, 1, rows1, gs1)
        _scale(rows1, eba, 1)
        _scatter(rows1, eba, 1, ss1)
        _swait(rows0, eba, 0, ss0)
        _gather(ebb, 0, rows0, gs0)
        _swait(rows1, eba, 1, ss1)
        _gather(ebb, 1, rows1, gs1)

        @pl.when(k < QPT - 1)
        def _():
            _stage(eba, q0 + 4)

        _gwait(ebb, 0, rows0, gs0)
        _scale(rows0, ebb, 0)
        _scatter(rows0, ebb, 0, ss0)
        _gwait(ebb, 1, rows1, gs1)
        _scale(rows1, ebb, 1)
        _scatter(rows1, ebb, 1, ss1)

        @pl.when(k < QPT - 1)
        def _():
            _swait(rows0, ebb, 0, ss0)
            _gather(eba, 0, rows0, gs0)
            _swait(rows1, ebb, 1, ss1)
            _gather(eba, 1, rows1, gs1)

        return ()

    lax.fori_loop(0, QPT, quad, ())
    _swait(rows0, ebb, 0, ss0)
    _swait(rows1, ebb, 1, ss1)
    plsc.subcore_barrier()
    pltpu.sync_copy(acc.at[pl.ds(s * NPS, NPS)],
                    out_hbm.at[c].at[pl.ds(s * NPS, NPS)])


def _t1_body(d0_ref, d1_ref, xp_ref, sc_ref, w1_ref, dis_ref, tbl_ref):
    d = d0_ref[0] + d1_ref[0] + 1.0
    dis = 1.0 / jnp.sqrt(d)
    dis_ref[0] = dis
    xs = xp_ref[0] * sc_ref[0]
    h = lax.dot_general(xs, w1_ref[...], (((1,), (1,)), ((), ())),
                        preferred_element_type=jnp.float32)
    t = h * dis
    tbl_ref[0, 0] = t[:, :64]
    tbl_ref[1, 0] = t[:, 64:]


def _t2_body(a0_ref, a1_ref, dis_ref, b1_ref, w2_ref, tbl_ref):
    a = jnp.concatenate([a0_ref[0], a1_ref[0]], axis=1)
    h1 = _lrelu(a * dis_ref[0] + b1_ref[...])
    hh = lax.dot_general(h1, w2_ref[...], (((1,), (1,)), ((), ())),
                         preferred_element_type=jnp.float32)
    t = hh * dis_ref[0]
    tbl_ref[0, 0] = t[:, :64]
    tbl_ref[1, 0] = t[:, 64:]


def _t34_body(a0_ref, a1_ref, dis_ref, b2_ref, p1w_ref, p1b_ref, g_ref,
              b_ref, p2w_ref, p2b_ref, o_ref, z_sc):
    i = pl.program_id(0)
    a = jnp.concatenate([a0_ref[0], a1_ref[0]], axis=1)
    h2 = _lrelu(a * dis_ref[0] + b2_ref[...])
    z_sc[i, :] = jnp.max(h2, axis=0)

    @pl.when(i == NUM_GRAPHS - 1)
    def _():
        p = lax.dot_general(z_sc[...], p1w_ref[...], (((1,), (1,)), ((), ())),
                            preferred_element_type=jnp.float32) + p1b_ref[...]
        mean = jnp.mean(p, axis=0, keepdims=True)
        var = jnp.mean((p - mean) ** 2, axis=0, keepdims=True)
        p = (p - mean) / jnp.sqrt(var + 1e-5) * g_ref[...] + b_ref[...]
        p = jnp.maximum(p, 0.0)
        o = lax.dot_general(p, p2w_ref[...], (((1,), (1,)), ((), ())),
                            preferred_element_type=jnp.float32) + p2b_ref[...]
        nrm = jnp.sqrt(jnp.sum(o * o, axis=1, keepdims=True))
        o_ref[...] = o / jnp.maximum(nrm, 1e-12)


def kernel(x, edge_index, edge_attr, batch, roi_scaler, W1, b1, W2, b2,
           P1w, P1b, gamma, beta, P2w, P2b):
    f32 = jnp.float32
    src32 = edge_index[0].reshape(NW, RPT32, CH)
    dst32 = edge_index[1].reshape(NW, RPT32, CH)
    w32 = edge_attr.reshape(NW, RPT32, CH)
    z1 = jnp.zeros((128,), f32)
    wbits = lax.bitcast_convert_type(edge_attr, jnp.int32)
    epk = jnp.concatenate([
        edge_index[0].reshape(EC64, 1, 64),
        edge_index[1].reshape(EC64, 1, 64),
        wbits.reshape(EC64, 1, 64)], axis=1)

    dp = _k1_deg(dst32, w32, z1)

    xp = jnp.pad(x, ((0, 0), (0, 13))).reshape(NS, NPS, 16)
    scp = jnp.pad(jnp.tile(roi_scaler, (NUM_GRAPHS, 1)),
                  ((0, 0), (0, 13))).reshape(NS, NPS, 16)
    w1p = jnp.pad(W1, ((0, 0), (0, 13)))

    blk_nps1 = pl.BlockSpec((1, NPS, 1), lambda i: (i, 0, 0))
    blk_nps16 = pl.BlockSpec((1, NPS, 16), lambda i: (i, 0, 0))
    blk_nps64 = pl.BlockSpec((1, NPS, 64), lambda i: (i, 0, 0))

    blk_tbl = pl.BlockSpec((2, 1, NPS, 64), lambda i: (0, i, 0, 0))
    dis, tblA = pl.pallas_call(
        _t1_body,
        grid=(NS,),
        in_specs=[
            blk_nps1, blk_nps1, blk_nps16, blk_nps16,
            pl.BlockSpec((128, 16), lambda i: (0, 0)),
        ],
        out_specs=[blk_nps1, blk_tbl],
        out_shape=[
            jax.ShapeDtypeStruct((NS, NPS, 1), f32),
            jax.ShapeDtypeStruct((2, NS, NPS, 64), f32),
        ],
    )(dp[0].reshape(NS, NPS, 1), dp[1].reshape(NS, NPS, 1), xp, scp, w1p)

    agg1 = _ka_agg(epk, tblA.reshape(2 * N, 64),
                   tblA.reshape(2, N, 64))

    tblB = pl.pallas_call(
        _t2_body,
        grid=(NS,),
        in_specs=[
            blk_nps64, blk_nps64, blk_nps1,
            pl.BlockSpec((1, 128), lambda i: (0, 0)),
            pl.BlockSpec((128, 128), lambda i: (0, 0)),
        ],
        out_specs=blk_tbl,
        out_shape=jax.ShapeDtypeStruct((2, NS, NPS, 64), f32),
    )(agg1[0].reshape(NS, NPS, 64), agg1[1].reshape(NS, NPS, 64),
      dis.reshape(NS, NPS, 1), b1.reshape(1, 128), W2)

    agg2 = _ka_agg(epk, tblB.reshape(2 * N, 64),
                   tblB.reshape(2, N, 64))

    full = lambda shp: pl.BlockSpec(shp, lambda i: tuple(0 for _ in shp))
    out = pl.pallas_call(
        _t34_body,
        grid=(NUM_GRAPHS,),
        in_specs=[
            pl.BlockSpec((1, NPG, 64), lambda i: (i, 0, 0)),
            pl.BlockSpec((1, NPG, 64), lambda i: (i, 0, 0)),
            pl.BlockSpec((1, NPG, 1), lambda i: (i, 0, 0)),
            full((1, 128)), full((512, 128)), full((1, 512)), full((1, 512)),
            full((1, 512)), full((1024, 512)), full((1, 1024)),
        ],
        out_specs=full((NUM_GRAPHS, 1024)),
        out_shape=jax.ShapeDtypeStruct((NUM_GRAPHS, 1024), f32),
        scratch_shapes=[pltpu.VMEM((NUM_GRAPHS, 128), f32)],
    )(agg2[0].reshape(NUM_GRAPHS, NPG, 64),
      agg2[1].reshape(NUM_GRAPHS, NPG, 64),
      dis.reshape(NUM_GRAPHS, NPG, 1), b2.reshape(1, 128),
      P1w, P1b.reshape(1, 512), gamma.reshape(1, 512), beta.reshape(1, 512),
      P2w, P2b.reshape(1, 1024))
    return out


# speedup vs baseline: 6.3597x; 1.0315x over previous
import functools

import jax
import jax.numpy as jnp
from jax import lax
from jax.experimental import pallas as pl
from jax.experimental.pallas import tpu as pltpu, tpu_sc as plsc

NUM_GRAPHS = 256
NPG = 113
N = NUM_GRAPHS * NPG
E = N * 16
NC, NS = 2, 16
NW = NC * NS
CH = 128
ER = E // CH
RPT32 = ER // NW
RPT16 = ER // NS
NPS = N // NS

_mesh = plsc.VectorSubcoreMesh(core_axis_name="c", subcore_axis_name="s")


def _lrelu(x):
    return jnp.where(x > 0, x, 0.2 * x)


@functools.partial(
    pl.kernel,
    out_type=jax.ShapeDtypeStruct((NC, N), jnp.float32),
    mesh=_mesh,
    compiler_params=pltpu.CompilerParams(use_tc_tiling_on_sc=False),
    scratch_types=[
        pltpu.VMEM((RPT32, CH), jnp.int32),
        pltpu.VMEM((RPT32, CH), jnp.float32),
        pltpu.VMEM_SHARED((N,), jnp.float32),
        pltpu.SemaphoreType.DMA,
    ],
)
def _k1_deg(dst_hbm, w_hbm, z1_hbm, out_hbm, dstb, wb, acc, sem1):
    c = lax.axis_index("c")
    s = lax.axis_index("s")
    wid = c * NS + s
    for t in range(15):
        j = s + NS * t

        @pl.when(j < N // 128)
        def _():
            pltpu.sync_copy(z1_hbm, acc.at[pl.ds(j * 128, 128)])

    plsc.subcore_barrier()
    pltpu.sync_copy(dst_hbm.at[wid], dstb)
    pltpu.sync_copy(w_hbm.at[wid], wb)

    def chunk(j, _):
        for u in range(8):
            pltpu.async_copy(wb.at[j * 8 + u], acc.at[dstb.at[j * 8 + u]],
                             sem1, add=True)
        for u in range(8):
            pltpu.make_async_copy(wb.at[j * 8 + u],
                                  acc.at[dstb.at[j * 8 + u]], sem1).wait()
        return ()

    lax.fori_loop(0, RPT32 // 8, chunk, ())
    pltpu.sync_copy(wb.at[RPT32 - 1], acc.at[dstb.at[RPT32 - 1]], add=True)
    plsc.subcore_barrier()
    for t in range(15):
        j = s + NS * t

        @pl.when(j < N // 128)
        def _():
            pltpu.sync_copy(acc.at[pl.ds(j * 128, 128)],
                            out_hbm.at[c].at[pl.ds(j * 128, 128)])


EC64 = E // 64
CPT = EC64 // NS
QPT = CPT // 4


@functools.partial(
    pl.kernel,
    out_type=jax.ShapeDtypeStruct((NC, N, 64), jnp.float32),
    mesh=_mesh,
    compiler_params=pltpu.CompilerParams(use_tc_tiling_on_sc=False,
                                         needs_layout_passes=False),
    scratch_types=[
        pltpu.VMEM((1, 3, 64), jnp.int32),
        pltpu.VMEM((1, 3, 64), jnp.int32),
        pltpu.VMEM((1, 3, 64), jnp.int32),
        pltpu.VMEM((64, 64), jnp.float32),
        pltpu.VMEM((64, 64), jnp.float32),
        pltpu.VMEM((64, 64), jnp.float32),
        pltpu.VMEM_SHARED((N, 64), jnp.float32),
        pltpu.SemaphoreType.DMA, pltpu.SemaphoreType.DMA,
        pltpu.SemaphoreType.DMA,
        pltpu.SemaphoreType.DMA, pltpu.SemaphoreType.DMA,
        pltpu.SemaphoreType.DMA,
        pltpu.SemaphoreType.DMA, pltpu.SemaphoreType.DMA,
        pltpu.SemaphoreType.DMA,
    ],
)
def _ka_agg(epk_hbm, tbl_hbm, gini_hbm, out_hbm,
            eb0, eb1, eb2, rows0, rows1, rows2, acc,
            gs0, gs1, gs2, ss0, ss1, ss2, st0, st1, st2):
    c = lax.axis_index("c")
    s = lax.axis_index("s")
    cn = c * N
    base = s * CPT
    slots = ((eb0, rows0, gs0, ss0, st0),
             (eb1, rows1, gs1, ss1, st1),
             (eb2, rows2, gs2, ss2, st2))

    pltpu.sync_copy(gini_hbm.at[c].at[pl.ds(s * NPS, NPS)],
                    acc.at[pl.ds(s * NPS, NPS)])
    plsc.subcore_barrier()

    def _transform(eb):
        for g in range(4):
            v = eb[0, 0, pl.ds(g * 16, 16)]
            eb[0, 0, pl.ds(g * 16, 16)] = v + cn

    def _scale(rows, eb):
        def grp(g, _):
            wv = plsc.bitcast(eb[0, 2, pl.ds(g * 16, 16)], jnp.float32)
            for l in range(16):
                e = g * 16 + l
                for q in range(4):
                    rows[e, pl.ds(q * 16, 16)] = (
                        rows[e, pl.ds(q * 16, 16)] * wv[l])
            return ()

        lax.fori_loop(0, 4, grp, ())

    for j, (eb, rows, gs, ss, st) in enumerate(slots):
        pltpu.sync_copy(epk_hbm.at[pl.ds(base + j, 1)], eb)
        _transform(eb)
        pltpu.async_copy(tbl_hbm.at[eb.at[0, 0]], rows, gs)

    def tri(k, _):
        a = base + 3 * k
        for j, (eb, rows, gs, ss, st) in enumerate(slots):
            pltpu.make_async_copy(tbl_hbm.at[eb.at[0, 0]], rows, gs).wait()
            _scale(rows, eb)
            pltpu.async_copy(rows, acc.at[eb.at[0, 1]], ss, add=True)
        for j, (eb, rows, gs, ss, st) in enumerate(slots):
            @pl.when(3 * k + j + 3 < CPT)
            def _():
                pltpu.make_async_copy(rows, acc.at[eb.at[0, 1]], ss).wait()
                pltpu.async_copy(epk_hbm.at[pl.ds(a + j + 3, 1)], eb, st)
        for j, (eb, rows, gs, ss, st) in enumerate(slots):
            @pl.when(3 * k + j + 3 < CPT)
            def _():
                pltpu.make_async_copy(epk_hbm.at[pl.ds(a + j + 3, 1)],
                                      eb, st).wait()
                _transform(eb)
                pltpu.async_copy(tbl_hbm.at[eb.at[0, 0]], rows, gs)
        return ()

    lax.fori_loop(0, (CPT - 2) // 3, tri, ())
    for j in range(2):
        eb, rows, gs, ss, st = slots[j]
        pltpu.make_async_copy(tbl_hbm.at[eb.at[0, 0]], rows, gs).wait()
        _scale(rows, eb)
        pltpu.async_copy(rows, acc.at[eb.at[0, 1]], ss, add=True)
    for j in range(3):
        eb, rows, gs, ss, st = slots[j]
        pltpu.make_async_copy(rows, acc.at[eb.at[0, 1]], ss).wait()
    plsc.subcore_barrier()
    pltpu.sync_copy(acc.at[pl.ds(s * NPS, NPS)],
                    out_hbm.at[c].at[pl.ds(s * NPS, NPS)])


def _t1_body(d0_ref, d1_ref, xp_ref, sc_ref, w1_ref, dis_ref, tbl_ref):
    d = d0_ref[0] + d1_ref[0] + 1.0
    dis = 1.0 / jnp.sqrt(d)
    dis_ref[0] = dis
    xs = xp_ref[0] * sc_ref[0]
    h = lax.dot_general(xs, w1_ref[...], (((1,), (1,)), ((), ())),
                        preferred_element_type=jnp.float32)
    t = h * dis
    tbl_ref[0, 0] = t[:, :64]
    tbl_ref[1, 0] = t[:, 64:]


def _t2_body(a0_ref, a1_ref, dis_ref, b1_ref, w2_ref, tbl_ref):
    a = jnp.concatenate([a0_ref[0], a1_ref[0]], axis=1)
    h1 = _lrelu(a * dis_ref[0] + b1_ref[...])
    hh = lax.dot_general(h1, w2_ref[...], (((1,), (1,)), ((), ())),
                         preferred_element_type=jnp.float32)
    t = hh * dis_ref[0]
    tbl_ref[0, 0] = t[:, :64]
    tbl_ref[1, 0] = t[:, 64:]


def _t34_body(a0_ref, a1_ref, dis_ref, b2_ref, p1w_ref, p1b_ref, g_ref,
              b_ref, p2w_ref, p2b_ref, o_ref, z_sc):
    i = pl.program_id(0)
    a = jnp.concatenate([a0_ref[0], a1_ref[0]], axis=1)
    h2 = _lrelu(a * dis_ref[0] + b2_ref[...])
    z_sc[i, :] = jnp.max(h2, axis=0)

    @pl.when(i == NUM_GRAPHS - 1)
    def _():
        p = lax.dot_general(z_sc[...], p1w_ref[...], (((1,), (1,)), ((), ())),
                            preferred_element_type=jnp.float32) + p1b_ref[...]
        mean = jnp.mean(p, axis=0, keepdims=True)
        var = jnp.mean((p - mean) ** 2, axis=0, keepdims=True)
        p = (p - mean) / jnp.sqrt(var + 1e-5) * g_ref[...] + b_ref[...]
        p = jnp.maximum(p, 0.0)
        o = lax.dot_general(p, p2w_ref[...], (((1,), (1,)), ((), ())),
                            preferred_element_type=jnp.float32) + p2b_ref[...]
        nrm = jnp.sqrt(jnp.sum(o * o, axis=1, keepdims=True))
        o_ref[...] = o / jnp.maximum(nrm, 1e-12)


def kernel(x, edge_index, edge_attr, batch, roi_scaler, W1, b1, W2, b2,
           P1w, P1b, gamma, beta, P2w, P2b):
    f32 = jnp.float32
    src32 = edge_index[0].reshape(NW, RPT32, CH)
    dst32 = edge_index[1].reshape(NW, RPT32, CH)
    w32 = edge_attr.reshape(NW, RPT32, CH)
    z1 = jnp.zeros((128,), f32)
    wbits = lax.bitcast_convert_type(edge_attr, jnp.int32)
    epk = jnp.concatenate([
        edge_index[0].reshape(EC64, 1, 64),
        edge_index[1].reshape(EC64, 1, 64),
        wbits.reshape(EC64, 1, 64)], axis=1)

    dp = _k1_deg(dst32, w32, z1)

    xp = jnp.pad(x, ((0, 0), (0, 13))).reshape(NS, NPS, 16)
    scp = jnp.pad(jnp.tile(roi_scaler, (NUM_GRAPHS, 1)),
                  ((0, 0), (0, 13))).reshape(NS, NPS, 16)
    w1p = jnp.pad(W1, ((0, 0), (0, 13)))

    blk_nps1 = pl.BlockSpec((1, NPS, 1), lambda i: (i, 0, 0))
    blk_nps16 = pl.BlockSpec((1, NPS, 16), lambda i: (i, 0, 0))
    blk_nps64 = pl.BlockSpec((1, NPS, 64), lambda i: (i, 0, 0))

    blk_tbl = pl.BlockSpec((2, 1, NPS, 64), lambda i: (0, i, 0, 0))
    dis, tblA = pl.pallas_call(
        _t1_body,
        grid=(NS,),
        in_specs=[
            blk_nps1, blk_nps1, blk_nps16, blk_nps16,
            pl.BlockSpec((128, 16), lambda i: (0, 0)),
        ],
        out_specs=[blk_nps1, blk_tbl],
        out_shape=[
            jax.ShapeDtypeStruct((NS, NPS, 1), f32),
            jax.ShapeDtypeStruct((2, NS, NPS, 64), f32),
        ],
    )(dp[0].reshape(NS, NPS, 1), dp[1].reshape(NS, NPS, 1), xp, scp, w1p)

    agg1 = _ka_agg(epk, tblA.reshape(2 * N, 64),
                   tblA.reshape(2, N, 64))

    tblB = pl.pallas_call(
        _t2_body,
        grid=(NS,),
        in_specs=[
            blk_nps64, blk_nps64, blk_nps1,
            pl.BlockSpec((1, 128), lambda i: (0, 0)),
            pl.BlockSpec((128, 128), lambda i: (0, 0)),
        ],
        out_specs=blk_tbl,
        out_shape=jax.ShapeDtypeStruct((2, NS, NPS, 64), f32),
    )(agg1[0].reshape(NS, NPS, 64), agg1[1].reshape(NS, NPS, 64),
      dis.reshape(NS, NPS, 1), b1.reshape(1, 128), W2)

    agg2 = _ka_agg(epk, tblB.reshape(2 * N, 64),
                   tblB.reshape(2, N, 64))

    full = lambda shp: pl.BlockSpec(shp, lambda i: tuple(0 for _ in shp))
    out = pl.pallas_call(
        _t34_body,
        grid=(NUM_GRAPHS,),
        in_specs=[
            pl.BlockSpec((1, NPG, 64), lambda i: (i, 0, 0)),
            pl.BlockSpec((1, NPG, 64), lambda i: (i, 0, 0)),
            pl.BlockSpec((1, NPG, 1), lambda i: (i, 0, 0)),
            full((1, 128)), full((512, 128)), full((1, 512)), full((1, 512)),
            full((1, 512)), full((1024, 512)), full((1, 1024)),
        ],
        out_specs=full((NUM_GRAPHS, 1024)),
        out_shape=jax.ShapeDtypeStruct((NUM_GRAPHS, 1024), f32),
        scratch_shapes=[pltpu.VMEM((NUM_GRAPHS, 128), f32)],
    )(agg2[0].reshape(NUM_GRAPHS, NPG, 64),
      agg2[1].reshape(NUM_GRAPHS, NPG, 64),
      dis.reshape(NUM_GRAPHS, NPG, 1), b2.reshape(1, 128),
      P1w, P1b.reshape(1, 512), gamma.reshape(1, 512), beta.reshape(1, 512),
      P2w, P2b.reshape(1, 1024))
    return out
